# Initial kernel scaffold; baseline (speedup 1.0000x reference)
import jax
import jax.numpy as jnp
from jax.experimental import pallas as pl


def kernel(x_user, x_item, edge_u2i, edge_i2u, W_e_u2i, b_e_u2i, W_e_i2u, b_e_i2u, W_n_user, b_n_user, W_n_item, b_n_item):
    raise NotImplementedError("write your pallas kernel here")


# speedup vs baseline: 3.8299x; 3.8299x over previous
import functools

import jax
import jax.numpy as jnp
from jax import lax
from jax.experimental import pallas as pl
from jax.experimental.pallas import tpu as pltpu
from jax.experimental.pallas import tpu_sc as plsc

_K = 128
_G = 16
_NS = 16
_NC = 2


def _msg_table_body(x_ref, w_ref, b_ref, y_ref):
    y_ref[0] = jnp.maximum(
        jnp.dot(x_ref[0], w_ref[0], preferred_element_type=jnp.float32)
        + b_ref[0], 0.0)


def _node_update_body(xu_ref, xi_ref, accu_ref, acci_ref, cntu_ref, cnti_ref,
                      wnu_ref, wni_ref, bnu_ref, bni_ref, ou_ref, oi_ref):
    d = xu_ref.shape[1]
    m_u = accu_ref[0] / jnp.maximum(cntu_ref[0][:, :1], 1.0)
    m_i = acci_ref[0] / jnp.maximum(cnti_ref[0][:, :1], 1.0)
    hu = (jnp.dot(xu_ref[...], wnu_ref[:d, :], preferred_element_type=jnp.float32)
          + jnp.dot(m_u, wnu_ref[d:, :], preferred_element_type=jnp.float32)
          + bnu_ref[...])
    hi = (jnp.dot(xi_ref[...], wni_ref[:d, :], preferred_element_type=jnp.float32)
          + jnp.dot(m_i, wni_ref[d:, :], preferred_element_type=jnp.float32)
          + bni_ref[...])
    ou_ref[...] = jnp.maximum(hu, 0.0)
    oi_ref[...] = jnp.maximum(hi, 0.0)


def _make_sc_segment_sum(n_pad, d, nch):
    rpt = n_pad // _NS
    assert rpt % _K == 0 and nch % _G == 0

    mesh = plsc.VectorSubcoreMesh(core_axis_name="c", subcore_axis_name="s")

    @functools.partial(
        pl.kernel, mesh=mesh,
        out_type=[
            jax.ShapeDtypeStruct((_NC * n_pad, d), jnp.float32),
            jax.ShapeDtypeStruct((_NC * n_pad, d), jnp.float32),
        ],
        scratch_types=[
            pltpu.VMEM((2 * _G, _K), jnp.int32),
            pltpu.VMEM((_K, d), jnp.float32),
            pltpu.VMEM((_K, d), jnp.float32),
            pltpu.VMEM_SHARED((n_pad, d), jnp.float32),
            pltpu.SemaphoreType.DMA,
        ],
    )
    def sc_seg(tab, src_r, dst_r, zeros_t, ones_t,
               acc_out, cnt_out, idxb, rows, ones_v, acc_sh, sem):
        cid = lax.axis_index("c")
        sid = lax.axis_index("s")
        r0 = sid * rpt
        o0 = cid * n_pad + r0
        c0 = (cid * _NS + sid) * nch
        sidx = idxb.at[pl.ds(0, _G)]
        didx = idxb.at[pl.ds(_G, _G)]

        pltpu.sync_copy(zeros_t, ones_v)
        for j in range(rpt // _K):
            pltpu.sync_copy(ones_v, acc_sh.at[pl.ds(r0 + j * _K, _K)])
        pltpu.sync_copy(ones_t, ones_v)
        plsc.subcore_barrier()

        for g0 in range(0, nch, _G):
            pltpu.sync_copy(src_r.at[pl.ds(c0 + g0, _G)], sidx)
            pltpu.sync_copy(dst_r.at[pl.ds(c0 + g0, _G)], didx)
            for j in range(_G):
                pltpu.async_copy(tab.at[sidx.at[j]], rows, sem).wait()
                pltpu.sync_copy(rows, acc_sh.at[didx.at[j]], add=True)
        plsc.subcore_barrier()

        for j in range(rpt // _K):
            pltpu.sync_copy(acc_sh.at[pl.ds(r0 + j * _K, _K)], rows)
            pltpu.sync_copy(rows, acc_out.at[pl.ds(o0 + j * _K, _K)])
        pltpu.sync_copy(zeros_t, rows)
        for j in range(rpt // _K):
            pltpu.sync_copy(rows, acc_sh.at[pl.ds(r0 + j * _K, _K)])
        plsc.subcore_barrier()

        for g0 in range(0, nch, _G):
            pltpu.sync_copy(dst_r.at[pl.ds(c0 + g0, _G)], didx)
            for j in range(_G):
                pltpu.sync_copy(ones_v, acc_sh.at[didx.at[j]], add=True)
        plsc.subcore_barrier()

        for j in range(rpt // _K):
            pltpu.sync_copy(acc_sh.at[pl.ds(r0 + j * _K, _K)], rows)
            pltpu.sync_copy(rows, cnt_out.at[pl.ds(o0 + j * _K, _K)])

    return sc_seg


def kernel(x_user, x_item, edge_u2i, edge_i2u,
           W_e_u2i, b_e_u2i, W_e_i2u, b_e_i2u,
           W_n_user, b_n_user, W_n_item, b_n_item):
    n_u, d = x_user.shape
    n_i = x_item.shape[0]
    e = edge_u2i.shape[1]
    assert n_u == n_i, "kernel assumes equal node counts per type"
    n = n_u

    blk = 1000
    nblk = n // blk

    x_st = jnp.stack([x_user, x_item])
    w_st = jnp.stack([W_e_u2i, W_e_i2u])
    b_st = jnp.stack([b_e_u2i, b_e_i2u]).reshape(2, 1, d)
    y = pl.pallas_call(
        _msg_table_body,
        grid=(2, nblk),
        in_specs=[
            pl.BlockSpec((1, blk, d), lambda t, i: (t, i, 0)),
            pl.BlockSpec((1, d, d), lambda t, i: (t, 0, 0)),
            pl.BlockSpec((1, 1, d), lambda t, i: (t, 0, 0)),
        ],
        out_specs=pl.BlockSpec((1, blk, d), lambda t, i: (t, i, 0)),
        out_shape=jax.ShapeDtypeStruct((2, n, d), jnp.float32),
    )(x_st, w_st, b_st)
    tab = y.reshape(2 * n, d)

    n_pad = ((n + _NS * _K - 1) // (_NS * _K)) * (_NS * _K)
    ept = _NS * _G * _K
    e_pad = ((e + ept - 1) // ept) * ept
    nch = e_pad // (_NS * _K)
    pad = e_pad - e

    e0 = edge_u2i.astype(jnp.int32)
    e1 = edge_i2u.astype(jnp.int32)
    src_r = jnp.concatenate([
        jnp.pad(e0[0], (0, pad)).reshape(-1, _K),
        (jnp.pad(e1[0], (0, pad)) + n).reshape(-1, _K),
    ])
    dst_r = jnp.concatenate([
        jnp.pad(e0[1], (0, pad), constant_values=n).reshape(-1, _K),
        jnp.pad(e1[1], (0, pad), constant_values=n).reshape(-1, _K),
    ])
    zeros_t = jnp.zeros((_K, d), jnp.float32)
    ones_t = jnp.ones((_K, d), jnp.float32)
    acc, cnt = _make_sc_segment_sum(n_pad, d, nch)(
        tab, src_r, dst_r, zeros_t, ones_t)
    acc = acc.reshape(_NC, n_pad, d)
    cnt = cnt.reshape(_NC, n_pad, d)

    out_user, out_item = pl.pallas_call(
        _node_update_body,
        grid=(nblk,),
        in_specs=[
            pl.BlockSpec((blk, d), lambda i: (i, 0)),
            pl.BlockSpec((blk, d), lambda i: (i, 0)),
            pl.BlockSpec((1, blk, d), lambda i: (1, i, 0)),
            pl.BlockSpec((1, blk, d), lambda i: (0, i, 0)),
            pl.BlockSpec((1, blk, d), lambda i: (1, i, 0)),
            pl.BlockSpec((1, blk, d), lambda i: (0, i, 0)),
            pl.BlockSpec((2 * d, d), lambda i: (0, 0)),
            pl.BlockSpec((2 * d, d), lambda i: (0, 0)),
            pl.BlockSpec((1, d), lambda i: (0, 0)),
            pl.BlockSpec((1, d), lambda i: (0, 0)),
        ],
        out_specs=[
            pl.BlockSpec((blk, d), lambda i: (i, 0)),
            pl.BlockSpec((blk, d), lambda i: (i, 0)),
        ],
        out_shape=[
            jax.ShapeDtypeStruct((n, d), jnp.float32),
            jax.ShapeDtypeStruct((n, d), jnp.float32),
        ],
    )(x_user, x_item, acc, acc, cnt, cnt,
      W_n_user, W_n_item, b_n_user.reshape(1, d), b_n_item.reshape(1, d))

    return (out_user, out_item)


# speedup vs baseline: 4.5961x; 1.2001x over previous
import functools

import jax
import jax.numpy as jnp
from jax import lax
from jax.experimental import pallas as pl
from jax.experimental.pallas import tpu as pltpu
from jax.experimental.pallas import tpu_sc as plsc

_K = 64
_G = 16
_NB = 4
_NS = 16
_NC = 2


def _msg_table_body(x_ref, w_ref, b_ref, y_ref):
    y_ref[0] = jnp.maximum(
        jnp.dot(x_ref[0], w_ref[0], preferred_element_type=jnp.float32)
        + b_ref[0], 0.0)


def _node_update_body(xu_ref, xi_ref, accu_ref, acci_ref, cntu_ref, cnti_ref,
                      wnu_ref, wni_ref, bnu_ref, bni_ref, ou_ref, oi_ref):
    d = xu_ref.shape[1]
    m_u = accu_ref[0] / jnp.maximum(cntu_ref[0][:, :1], 1.0)
    m_i = acci_ref[0] / jnp.maximum(cnti_ref[0][:, :1], 1.0)
    hu = (jnp.dot(xu_ref[...], wnu_ref[:d, :], preferred_element_type=jnp.float32)
          + jnp.dot(m_u, wnu_ref[d:, :], preferred_element_type=jnp.float32)
          + bnu_ref[...])
    hi = (jnp.dot(xi_ref[...], wni_ref[:d, :], preferred_element_type=jnp.float32)
          + jnp.dot(m_i, wni_ref[d:, :], preferred_element_type=jnp.float32)
          + bni_ref[...])
    ou_ref[...] = jnp.maximum(hu, 0.0)
    oi_ref[...] = jnp.maximum(hi, 0.0)


def _make_sc_segment_sum(n_pad, d, nch):
    rpt = n_pad // _NS
    assert rpt % _K == 0 and nch % _G == 0

    mesh = plsc.VectorSubcoreMesh(core_axis_name="c", subcore_axis_name="s")

    @functools.partial(
        pl.kernel, mesh=mesh,
        out_type=[
            jax.ShapeDtypeStruct((_NC * n_pad, d), jnp.float32),
            jax.ShapeDtypeStruct((_NC * n_pad, d), jnp.float32),
        ],
        scratch_types=[
            pltpu.VMEM((2 * _G, _K), jnp.int32),
            pltpu.VMEM((_NB, _K, d), jnp.float32),
            pltpu.VMEM_SHARED((n_pad, d), jnp.float32),
            pltpu.SemaphoreType.DMA,
            pltpu.SemaphoreType.DMA,
            pltpu.SemaphoreType.DMA,
            pltpu.SemaphoreType.DMA,
            pltpu.SemaphoreType.DMA,
            pltpu.SemaphoreType.DMA,
            pltpu.SemaphoreType.DMA,
            pltpu.SemaphoreType.DMA,
            pltpu.SemaphoreType.DMA,
        ],
    )
    def sc_seg(tab, src_r, dst_r, zeros_t, ones_t,
               acc_out, cnt_out, idxb, rowsb, acc_sh,
               sg0, sg1, sg2, sg3, ss0, ss1, ss2, ss3, sem):
        rows = rowsb.at[0]
        aux = rowsb.at[1]
        semg = (sg0, sg1, sg2, sg3)
        sems = (ss0, ss1, ss2, ss3)
        cid = lax.axis_index("c")
        sid = lax.axis_index("s")
        r0 = sid * rpt
        o0 = cid * n_pad + r0
        c0 = (cid * _NS + sid) * nch
        sidx = idxb.at[pl.ds(0, _G)]
        didx = idxb.at[pl.ds(_G, _G)]

        pltpu.sync_copy(zeros_t, aux)
        for j in range(rpt // _K):
            pltpu.sync_copy(aux, acc_sh.at[pl.ds(r0 + j * _K, _K)])
        plsc.subcore_barrier()

        for g0 in range(0, nch, _G):
            pltpu.sync_copy(src_r.at[pl.ds(c0 + g0, _G)], sidx)
            pltpu.sync_copy(dst_r.at[pl.ds(c0 + g0, _G)], didx)
            hg = [None] * _G
            hs = [None] * _G
            for j in range(_NB - 1):
                hg[j] = pltpu.async_copy(
                    tab.at[sidx.at[j]], rowsb.at[j % _NB], semg[j % _NB])
            for j in range(_G):
                b = j % _NB
                hg[j].wait()
                hs[j] = pltpu.async_copy(
                    rowsb.at[b], acc_sh.at[didx.at[j]], sems[b], add=True)
                if j + _NB - 1 < _G:
                    nb = (j + _NB - 1) % _NB
                    if j >= 1:
                        hs[j - 1].wait()
                    hg[j + _NB - 1] = pltpu.async_copy(
                        tab.at[sidx.at[j + _NB - 1]], rowsb.at[nb], semg[nb])
            for j in range(max(0, _G - _NB), _G):
                hs[j].wait()
        plsc.subcore_barrier()

        pltpu.sync_copy(zeros_t, aux)
        for j in range(rpt // _K):
            pltpu.sync_copy(acc_sh.at[pl.ds(r0 + j * _K, _K)], rows)
            pltpu.sync_copy(rows, acc_out.at[pl.ds(o0 + j * _K, _K)])
            pltpu.sync_copy(aux, acc_sh.at[pl.ds(r0 + j * _K, _K)])
        pltpu.sync_copy(ones_t, aux)
        plsc.subcore_barrier()

        for g0 in range(0, nch, _G):
            pltpu.sync_copy(dst_r.at[pl.ds(c0 + g0, _G)], didx)
            hs = [pltpu.async_copy(aux, acc_sh.at[didx.at[j]], sem,
                                   add=True)
                  for j in range(_G)]
            for h in hs:
                h.wait()
        plsc.subcore_barrier()

        for j in range(rpt // _K):
            pltpu.sync_copy(acc_sh.at[pl.ds(r0 + j * _K, _K)], rows)
            pltpu.sync_copy(rows, cnt_out.at[pl.ds(o0 + j * _K, _K)])

    return sc_seg


def kernel(x_user, x_item, edge_u2i, edge_i2u,
           W_e_u2i, b_e_u2i, W_e_i2u, b_e_i2u,
           W_n_user, b_n_user, W_n_item, b_n_item):
    n_u, d = x_user.shape
    n_i = x_item.shape[0]
    e = edge_u2i.shape[1]
    assert n_u == n_i, "kernel assumes equal node counts per type"
    n = n_u

    blk = 1000
    nblk = n // blk

    x_st = jnp.stack([x_user, x_item])
    w_st = jnp.stack([W_e_u2i, W_e_i2u])
    b_st = jnp.stack([b_e_u2i, b_e_i2u]).reshape(2, 1, d)
    y = pl.pallas_call(
        _msg_table_body,
        grid=(2, nblk),
        in_specs=[
            pl.BlockSpec((1, blk, d), lambda t, i: (t, i, 0)),
            pl.BlockSpec((1, d, d), lambda t, i: (t, 0, 0)),
            pl.BlockSpec((1, 1, d), lambda t, i: (t, 0, 0)),
        ],
        out_specs=pl.BlockSpec((1, blk, d), lambda t, i: (t, i, 0)),
        out_shape=jax.ShapeDtypeStruct((2, n, d), jnp.float32),
    )(x_st, w_st, b_st)
    tab = y.reshape(2 * n, d)

    n_pad = ((n + _NS * _K - 1) // (_NS * _K)) * (_NS * _K)
    ept = _NS * _G * _K
    e_pad = ((e + ept - 1) // ept) * ept
    nch = e_pad // (_NS * _K)
    pad = e_pad - e

    e0 = edge_u2i.astype(jnp.int32)
    e1 = edge_i2u.astype(jnp.int32)
    src_r = jnp.concatenate([
        jnp.pad(e0[0], (0, pad)).reshape(-1, _K),
        (jnp.pad(e1[0], (0, pad)) + n).reshape(-1, _K),
    ])
    dst_r = jnp.concatenate([
        jnp.pad(e0[1], (0, pad), constant_values=n).reshape(-1, _K),
        jnp.pad(e1[1], (0, pad), constant_values=n).reshape(-1, _K),
    ])
    zeros_t = jnp.zeros((_K, d), jnp.float32)
    ones_t = jnp.ones((_K, d), jnp.float32)
    acc, cnt = _make_sc_segment_sum(n_pad, d, nch)(
        tab, src_r, dst_r, zeros_t, ones_t)
    acc = acc.reshape(_NC, n_pad, d)
    cnt = cnt.reshape(_NC, n_pad, d)

    out_user, out_item = pl.pallas_call(
        _node_update_body,
        grid=(nblk,),
        in_specs=[
            pl.BlockSpec((blk, d), lambda i: (i, 0)),
            pl.BlockSpec((blk, d), lambda i: (i, 0)),
            pl.BlockSpec((1, blk, d), lambda i: (1, i, 0)),
            pl.BlockSpec((1, blk, d), lambda i: (0, i, 0)),
            pl.BlockSpec((1, blk, d), lambda i: (1, i, 0)),
            pl.BlockSpec((1, blk, d), lambda i: (0, i, 0)),
            pl.BlockSpec((2 * d, d), lambda i: (0, 0)),
            pl.BlockSpec((2 * d, d), lambda i: (0, 0)),
            pl.BlockSpec((1, d), lambda i: (0, 0)),
            pl.BlockSpec((1, d), lambda i: (0, 0)),
        ],
        out_specs=[
            pl.BlockSpec((blk, d), lambda i: (i, 0)),
            pl.BlockSpec((blk, d), lambda i: (i, 0)),
        ],
        out_shape=[
            jax.ShapeDtypeStruct((n, d), jnp.float32),
            jax.ShapeDtypeStruct((n, d), jnp.float32),
        ],
    )(x_user, x_item, acc, acc, cnt, cnt,
      W_n_user, W_n_item, b_n_user.reshape(1, d), b_n_item.reshape(1, d))

    return (out_user, out_item)
